# baseline (speedup 1.0000x reference)
import functools

import numpy as np
import jax
import jax.numpy as jnp
from jax.experimental import pallas as pl

_B, _N, _D, _R = 1, 2048, 1024, 8
_CB, _CIC, _KV = 64, 16, 64
_OP = (_D * _R) // _CB
_G = 8
_T = 512

_PERM = np.arange(_CB).reshape(8, 8).T.reshape(-1)


def _prep(W, keys, vals):
    Wt = W.reshape(_CB * _CIC, _D)
    kp = keys.reshape(_G, 8, _KV, _CIC)
    eye = jnp.eye(8, dtype=W.dtype)
    KT = jnp.einsum('hjkg,ji->hjkig', kp, eye).reshape(_G, 8 * _KV, 8 * _CIC)
    kn = (kp ** 2).sum(-1).reshape(_G, 8 * _KV, 1)
    V = vals[_PERM].reshape(_G, 8 * _KV, _OP)
    return Wt, KT, kn, V


def _kmin(sc, ko):
    m = jnp.min(sc, axis=1, keepdims=True)
    return jnp.min(jnp.where(sc == m, ko, _KV), axis=1, keepdims=True)


def _body(x_ref, wa_ref, ka_ref, kna_ref, va_ref,
          wb_ref, kb_ref, knb_ref, vbt_ref, out_ref):
    xt = jnp.transpose(x_ref[...])
    pTA = jnp.dot(wa_ref[...], xt, preferred_element_type=jnp.float32)
    pTB = jnp.dot(wb_ref[...], xt, preferred_element_type=jnp.float32)
    ko = jax.lax.broadcasted_iota(jnp.int32, (8, _KV, _T), 1)
    kmA = [None] * _G
    kmB = [None] * _G
    for h in range(_G):
        pa = pTA[h * 128:(h + 1) * 128, :]
        pnA = jnp.sum((pa * pa).reshape(8, _CIC, _T), axis=1, keepdims=True)
        crossA = jnp.dot(ka_ref[h], pa, preferred_element_type=jnp.float32)
        d2A = (pnA + kna_ref[h].reshape(8, _KV, 1)) \
            - 2.0 * crossA.reshape(8, _KV, _T)
        kmA[h] = _kmin(jnp.sqrt(jnp.maximum(d2A, 0.0)), ko)
        pb = pTB[h * 128:(h + 1) * 128, :]
        pnB = jnp.sum((pb * pb).reshape(8, _CIC, _T), axis=1, keepdims=True)
        crossB = jnp.dot(kb_ref[h], pb, preferred_element_type=jnp.float32)
        d2B = (pnB + knb_ref[h].reshape(8, _KV, 1)) \
            - 2.0 * crossB.reshape(8, _KV, _T)
        kmB[h] = _kmin(jnp.sqrt(jnp.maximum(d2B, 0.0)), ko)
    t = None
    kBq = [None] * _G
    for q in range(_G):
        kAq = jnp.concatenate([kmA[r][q:q + 1] for r in range(8)], axis=0)
        kBq[q] = jnp.concatenate([kmB[r][q:q + 1] for r in range(8)], axis=0)
        PT = jnp.dot(va_ref[q], xt[q * 128:(q + 1) * 128, :],
                     preferred_element_type=jnp.float32).reshape(8, _KV, _T)
        s = jnp.sum(jnp.where(ko == kAq, PT, 0.0), axis=1, keepdims=True)
        t = s if t is None else t + s
    for q in range(_G):
        w = jnp.where(ko == kBq[q], jnp.broadcast_to(t, ko.shape), 0.0)
        out_ref[:, q * 128:(q + 1) * 128] = jnp.transpose(jnp.dot(
            vbt_ref[q], w.reshape(8 * _KV, _T),
            preferred_element_type=jnp.float32))


@functools.partial(jax.jit, static_argnames=("interpret",))
def _run(x, W_A, keys_A, vals_A, W_B, keys_B, vals_B, interpret=False):
    WAt, KAT, knA, VA = _prep(W_A, keys_A, vals_A)
    WBt, KBT, knB, VB = _prep(W_B, keys_B, vals_B)
    VBT = VB.transpose(0, 2, 1)
    full = lambda *s: pl.BlockSpec(s, lambda i: (0,) * len(s))
    out = pl.pallas_call(
        _body,
        grid=(_N // _T,),
        in_specs=[
            pl.BlockSpec((_T, _D), lambda i: (i, 0)),
            full(_CB * _CIC, _D),
            full(_G, 8 * _KV, 8 * _CIC),
            full(_G, 8 * _KV, 1),
            full(_G, 8 * _KV, _OP),
            full(_CB * _CIC, _D),
            full(_G, 8 * _KV, 8 * _CIC),
            full(_G, 8 * _KV, 1),
            full(_G, _OP, 8 * _KV),
        ],
        out_specs=pl.BlockSpec((_T, _D), lambda i: (i, 0)),
        out_shape=jax.ShapeDtypeStruct((_N, _D), jnp.float32),
        interpret=interpret,
    )(x.reshape(_N, _D), WAt, KAT, knA, VA, WBt, KBT, knB, VBT)
    return out.reshape(_B, _N, _D)


def kernel(x, W_A, keys_A, vals_A, W_B, keys_B, vals_B):
    return _run(x, W_A, keys_A, vals_A, W_B, keys_B, vals_B)
